# baseline (speedup 1.0000x reference)
import functools

import jax
import jax.numpy as jnp
from jax import lax
from jax.experimental import pallas as pl
from jax.experimental.pallas import tpu as pltpu
from jax.experimental.pallas import tpu_sc as plsc

_NC = 2
_NS = 16
_LANES = 16

_D = 64
_NW32 = _D // 32
_ND = _D // _LANES
_NPHRASE = 3
_L = 20
_K = _NPHRASE * _L

_CB = 16
_SEG = 120
_NSEG = (_CB * _K) // _SEG


def _allsum(v, lane):
    for sh in (1, 2, 4, 8):
        v = v + v.at[lane ^ sh].get(mode="promise_in_bounds")
    return v


def _rsqrt_vec(x):
    bits = lax.bitcast_convert_type(x, jnp.int32)
    y = lax.bitcast_convert_type(jnp.int32(0x5F3759DF) - (bits >> 1), jnp.float32)
    for _ in range(3):
        y = y * (1.5 - 0.5 * x * y * y)
    return y


def _sc_kernel(bpw, idx_hbm, table_hbm, params_hbm, out_hbm,
               idx_v, rows_v, params_v, out_v, sem0, sem1):
    wid = lax.axis_index("s") * _NC + lax.axis_index("c")
    nch = bpw // _CB
    sems = (sem0, sem1)

    pltpu.sync_copy(idx_hbm.at[pl.ds(wid * nch, nch)], idx_v)
    pltpu.sync_copy(params_hbm, params_v)
    tsum = [params_v[0, pl.ds(d * _LANES, _LANES)]
            + params_v[1, pl.ds(d * _LANES, _LANES)]
            + params_v[2, pl.ds(d * _LANES, _LANES)] for d in range(_ND)]
    gam = [params_v[3, pl.ds(d * _LANES, _LANES)] for d in range(_ND)]
    bet = [params_v[4, pl.ds(d * _LANES, _LANES)] for d in range(_ND)]

    def gather_descs(g, buf):
        return [pltpu.make_async_copy(
                    table_hbm.at[idx_v.at[g, s]],
                    rows_v.at[buf, pl.ds(s * _SEG, _SEG)],
                    sems[buf]) for s in range(_NSEG)]

    def issue(g, buf):
        for cp in gather_descs(g, buf):
            cp.start()

    issue(0, 0)
    lane = lax.iota(jnp.int32, _LANES)
    mask_hi = jnp.full((_LANES,), jnp.int32(-65536))

    def outer(g2, carry):
        for par in range(2):
            g = g2 * 2 + par

            @pl.when(g + 1 < nch)
            def _():
                issue(g + 1, 1 - par)

            for cp in gather_descs(g, par):
                cp.wait()

            for b in range(_CB):
                def row_body(j, acc):
                    r = b * _K + j
                    a = list(acc)
                    for h in range(_NW32):
                        w = rows_v[par, r, pl.ds(h * _LANES, _LANES)]
                        lo = lax.bitcast_convert_type(
                            lax.shift_left(w, 16), jnp.float32)
                        hi = lax.bitcast_convert_type(w & mask_hi, jnp.float32)
                        a[h] = a[h] + lo
                        a[2 + h] = a[2 + h] + hi
                    return tuple(a)

                zero = jnp.zeros((_LANES,), jnp.float32)
                acc = lax.fori_loop(0, _K, row_body, (zero,) * _ND, unroll=6)
                y = [acc[d] * (1.0 / _L) + tsum[d] for d in range(_ND)]

                s4 = (y[0] + y[1]) + (y[2] + y[3])
                mu = _allsum(s4, lane) * (1.0 / _D)
                xc = [y[d] - mu for d in range(_ND)]
                q = (xc[0] * xc[0] + xc[1] * xc[1]) + (xc[2] * xc[2] + xc[3] * xc[3])
                var = _allsum(q, lane) * (1.0 / _D)
                rstd = _rsqrt_vec(var + 1e-5)
                row = g * _CB + b
                for d in range(_ND):
                    out_v[row, pl.ds(d * _LANES, _LANES)] = (
                        xc[d] * rstd * gam[d] + bet[d])
        return carry

    lax.fori_loop(0, nch // 2, outer, 0)
    pltpu.sync_copy(out_v, out_hbm.at[pl.ds(wid * bpw, bpw), :])


def kernel(action_input, arg1_input, arg2_input, emb_table, type_table,
           ln_gamma, ln_beta):
    b = action_input.shape[0]
    v = emb_table.shape[0]
    nw = _NC * _NS
    bpw = b // nw
    idx = jnp.concatenate([action_input, arg1_input, arg2_input], axis=1)
    idx = idx.astype(jnp.int32).reshape(b // _CB, _NSEG, _SEG)
    flat = emb_table.reshape(v * _D // 128, 128)

    def _pk(x, y):
        xb = lax.bitcast_convert_type(x.astype(jnp.bfloat16), jnp.uint16)
        yb = lax.bitcast_convert_type(y.astype(jnp.bfloat16), jnp.uint16)
        return xb.astype(jnp.int32) | (yb.astype(jnp.int32) << 16)

    blocks = [_pk(flat[(k // 2)::2, (k % 2) * 64:(k % 2) * 64 + 32],
                  flat[(k // 2)::2, (k % 2) * 64 + 32:(k % 2) * 64 + 64])
              for k in range(4)]
    table32 = jnp.concatenate(blocks, axis=1).reshape(v, _D // 2)
    params = jnp.concatenate(
        [type_table.astype(jnp.float32),
         ln_gamma.astype(jnp.float32)[None, :],
         ln_beta.astype(jnp.float32)[None, :]], axis=0)

    mesh = plsc.VectorSubcoreMesh(core_axis_name="c", subcore_axis_name="s")
    run = pl.kernel(
        functools.partial(_sc_kernel, bpw),
        mesh=mesh,
        compiler_params=pltpu.CompilerParams(use_tc_tiling_on_sc=False),
        out_type=jax.ShapeDtypeStruct((b, _D), jnp.float32),
        scratch_types=[
            pltpu.VMEM((bpw // _CB, _NSEG, _SEG), jnp.int32),
            pltpu.VMEM((2, _CB * _K, _D // 2), jnp.int32),
            pltpu.VMEM((5, _D), jnp.float32),
            pltpu.VMEM((bpw, _D), jnp.float32),
            pltpu.SemaphoreType.DMA,
            pltpu.SemaphoreType.DMA,
        ],
    )
    return run(idx, table32, params)


# speedup vs baseline: 2.5418x; 2.5418x over previous
import functools

import jax
import jax.numpy as jnp
from jax import lax
from jax.experimental import pallas as pl
from jax.experimental.pallas import tpu as pltpu
from jax.experimental.pallas import tpu_sc as plsc

_NC = 2
_NS = 16
_LANES = 16

_D = 64
_NW32 = _D // 32
_ND = _D // _LANES
_NPHRASE = 3
_L = 20
_K = _NPHRASE * _L

_CB = 16
_SEG = 120
_NSEG = (_CB * _K) // _SEG


def _allsum(v, lane):
    for sh in (1, 2, 4, 8):
        v = v + v.at[lane ^ sh].get(mode="promise_in_bounds")
    return v


def _rsqrt_vec(x):
    bits = lax.bitcast_convert_type(x, jnp.int32)
    y = lax.bitcast_convert_type(jnp.int32(0x5F3759DF) - (bits >> 1), jnp.float32)
    for _ in range(3):
        y = y * (1.5 - 0.5 * x * y * y)
    return y


def _sc_kernel(bpw, idx_hbm, table_hbm, params_hbm, out_hbm,
               idx_v, rows_v, params_v, out_v, sem0, sem1):
    wid = lax.axis_index("s") * _NC + lax.axis_index("c")
    nch = bpw // _CB
    sems = (sem0, sem1)

    pltpu.sync_copy(idx_hbm.at[pl.ds(wid * nch, nch)], idx_v)
    pltpu.sync_copy(params_hbm, params_v)
    tsum = [params_v[0, pl.ds(d * _LANES, _LANES)]
            + params_v[1, pl.ds(d * _LANES, _LANES)]
            + params_v[2, pl.ds(d * _LANES, _LANES)] for d in range(_ND)]
    gam = [params_v[3, pl.ds(d * _LANES, _LANES)] for d in range(_ND)]
    bet = [params_v[4, pl.ds(d * _LANES, _LANES)] for d in range(_ND)]

    def gather_descs(g, buf):
        return [pltpu.make_async_copy(
                    table_hbm.at[idx_v.at[g, s]],
                    rows_v.at[buf, pl.ds(s * _SEG, _SEG)],
                    sems[buf]) for s in range(_NSEG)]

    def issue(g, buf):
        for cp in gather_descs(g, buf):
            cp.start()

    issue(0, 0)
    lane = lax.iota(jnp.int32, _LANES)
    mask_hi = jnp.full((_LANES,), jnp.int32(-65536))

    def outer(g2, carry):
        for par in range(2):
            g = g2 * 2 + par

            @pl.when(g + 1 < nch)
            def _():
                issue(g + 1, 1 - par)

            for cp in gather_descs(g, par):
                cp.wait()

            for b in range(_CB):
                def row_body(j, acc):
                    r = b * _K + j
                    a = list(acc)
                    for h in range(_NW32):
                        w = rows_v[par, r, pl.ds(h * _LANES, _LANES)]
                        lo = lax.bitcast_convert_type(
                            lax.shift_left(w, 16), jnp.float32)
                        hi = lax.bitcast_convert_type(w & mask_hi, jnp.float32)
                        a[h] = a[h] + lo
                        a[2 + h] = a[2 + h] + hi
                    return tuple(a)

                zero = jnp.zeros((_LANES,), jnp.float32)
                acc = lax.fori_loop(0, _K, row_body, (zero,) * _ND, unroll=6)
                y = [acc[d] * (1.0 / _L) + tsum[d] for d in range(_ND)]

                s4 = (y[0] + y[1]) + (y[2] + y[3])
                mu = _allsum(s4, lane) * (1.0 / _D)
                xc = [y[d] - mu for d in range(_ND)]
                q = (xc[0] * xc[0] + xc[1] * xc[1]) + (xc[2] * xc[2] + xc[3] * xc[3])
                var = _allsum(q, lane) * (1.0 / _D)
                rstd = _rsqrt_vec(var + 1e-5)
                row = g * _CB + b
                for d in range(_ND):
                    out_v[row, pl.ds(d * _LANES, _LANES)] = (
                        xc[d] * rstd * gam[d] + bet[d])
        return carry

    lax.fori_loop(0, nch // 2, outer, 0)
    pltpu.sync_copy(out_v, out_hbm.at[pl.ds(wid * bpw, bpw), :])


def kernel(action_input, arg1_input, arg2_input, emb_table, type_table,
           ln_gamma, ln_beta):
    b = action_input.shape[0]
    v = emb_table.shape[0]
    nw = _NC * _NS
    bpw = b // nw
    idx = jnp.concatenate([action_input, arg1_input, arg2_input], axis=1)
    idx = idx.astype(jnp.int32).reshape(b // _CB, _NSEG, _SEG)
    bits = lax.bitcast_convert_type(emb_table, jnp.uint32)
    rnd = (bits + 0x7FFF + ((bits >> 16) & 1)) >> 16
    table32 = lax.bitcast_convert_type(
        rnd[:, :32] | (rnd[:, 32:] << 16), jnp.int32)
    params = jnp.concatenate(
        [type_table.astype(jnp.float32),
         ln_gamma.astype(jnp.float32)[None, :],
         ln_beta.astype(jnp.float32)[None, :]], axis=0)

    mesh = plsc.VectorSubcoreMesh(core_axis_name="c", subcore_axis_name="s")
    run = pl.kernel(
        functools.partial(_sc_kernel, bpw),
        mesh=mesh,
        compiler_params=pltpu.CompilerParams(use_tc_tiling_on_sc=False),
        out_type=jax.ShapeDtypeStruct((b, _D), jnp.float32),
        scratch_types=[
            pltpu.VMEM((bpw // _CB, _NSEG, _SEG), jnp.int32),
            pltpu.VMEM((2, _CB * _K, _D // 2), jnp.int32),
            pltpu.VMEM((5, _D), jnp.float32),
            pltpu.VMEM((bpw, _D), jnp.float32),
            pltpu.SemaphoreType.DMA,
            pltpu.SemaphoreType.DMA,
        ],
    )
    return run(idx, table32, params)


# speedup vs baseline: 3.1128x; 1.2246x over previous
import functools

import jax
import jax.numpy as jnp
from jax import lax
from jax.experimental import pallas as pl
from jax.experimental.pallas import tpu as pltpu
from jax.experimental.pallas import tpu_sc as plsc

_NC = 2
_NS = 16
_LANES = 16

_D = 64
_ND = _D // _LANES
_NPHRASE = 3
_L = 20
_K = _NPHRASE * _L

_CB = 8
_SEG = 120
_NSEG = (_CB * _K) // _SEG


def _allsum(v, lane):
    for sh in (1, 2, 4, 8):
        v = v + v.at[lane ^ sh].get(mode="promise_in_bounds")
    return v


def _rsqrt_vec(x):
    bits = lax.bitcast_convert_type(x, jnp.int32)
    y = lax.bitcast_convert_type(jnp.int32(0x5F3759DF) - (bits >> 1), jnp.float32)
    for _ in range(3):
        y = y * (1.5 - 0.5 * x * y * y)
    return y


def _sc_kernel(bpw, idx_hbm, table_hbm, params_hbm, out_hbm,
               idx_v, rows_v, params_v, out_v, sem0, sem1):
    wid = lax.axis_index("s") * _NC + lax.axis_index("c")
    nch = bpw // _CB
    sems = (sem0, sem1)

    pltpu.sync_copy(idx_hbm.at[pl.ds(wid * nch, nch)], idx_v)
    pltpu.sync_copy(params_hbm, params_v)
    tsum = [params_v[0, pl.ds(d * _LANES, _LANES)]
            + params_v[1, pl.ds(d * _LANES, _LANES)]
            + params_v[2, pl.ds(d * _LANES, _LANES)] for d in range(_ND)]
    gam = [params_v[3, pl.ds(d * _LANES, _LANES)] for d in range(_ND)]
    bet = [params_v[4, pl.ds(d * _LANES, _LANES)] for d in range(_ND)]

    def gather_descs(g, buf):
        return [pltpu.make_async_copy(
                    table_hbm.at[idx_v.at[g, s]],
                    rows_v.at[buf, pl.ds(s * _SEG, _SEG)],
                    sems[buf]) for s in range(_NSEG)]

    def issue(g, buf):
        for cp in gather_descs(g, buf):
            cp.start()

    issue(0, 0)

    def outer(g2, carry):
        for par in range(2):
            g = g2 * 2 + par

            @pl.when(g + 1 < nch)
            def _():
                issue(g + 1, 1 - par)

            for cp in gather_descs(g, par):
                cp.wait()

            for b in range(_CB):
                def row_body(j, acc):
                    r = b * _K + j
                    return tuple(acc[d] + rows_v[par, r, pl.ds(d * _LANES, _LANES)]
                                 for d in range(_ND))

                zero = jnp.zeros((_LANES,), jnp.float32)
                acc = lax.fori_loop(0, _K, row_body, (zero,) * _ND, unroll=12)
                y = [acc[d] * (1.0 / _L) + tsum[d] for d in range(_ND)]

                lane = lax.iota(jnp.int32, _LANES)
                s4 = (y[0] + y[1]) + (y[2] + y[3])
                q4 = (y[0] * y[0] + y[1] * y[1]) + (y[2] * y[2] + y[3] * y[3])
                mu = _allsum(s4, lane) * (1.0 / _D)
                var = _allsum(q4, lane) * (1.0 / _D) - mu * mu
                rstd = _rsqrt_vec(var + 1e-5)
                row = g * _CB + b
                for d in range(_ND):
                    gd = rstd * gam[d]
                    out_v[row, pl.ds(d * _LANES, _LANES)] = (
                        (y[d] - mu) * gd + bet[d])
        return carry

    lax.fori_loop(0, nch // 2, outer, 0)
    pltpu.sync_copy(out_v, out_hbm.at[pl.ds(wid * bpw, bpw), :])


def kernel(action_input, arg1_input, arg2_input, emb_table, type_table,
           ln_gamma, ln_beta):
    b = action_input.shape[0]
    nw = _NC * _NS
    bpw = b // nw
    idx = jnp.concatenate([action_input, arg1_input, arg2_input], axis=1)
    idx = idx.astype(jnp.int32).reshape(b // _CB, _NSEG, _SEG)
    params = jnp.concatenate(
        [type_table.astype(jnp.float32),
         ln_gamma.astype(jnp.float32)[None, :],
         ln_beta.astype(jnp.float32)[None, :]], axis=0)

    mesh = plsc.VectorSubcoreMesh(core_axis_name="c", subcore_axis_name="s")
    run = pl.kernel(
        functools.partial(_sc_kernel, bpw),
        mesh=mesh,
        compiler_params=pltpu.CompilerParams(use_tc_tiling_on_sc=False),
        out_type=jax.ShapeDtypeStruct((b, _D), jnp.float32),
        scratch_types=[
            pltpu.VMEM((bpw // _CB, _NSEG, _SEG), jnp.int32),
            pltpu.VMEM((2, _CB * _K, _D), jnp.float32),
            pltpu.VMEM((5, _D), jnp.float32),
            pltpu.VMEM((bpw, _D), jnp.float32),
            pltpu.SemaphoreType.DMA,
            pltpu.SemaphoreType.DMA,
        ],
    )
    return run(idx, emb_table, params)


# speedup vs baseline: 3.4293x; 1.1017x over previous
import functools

import jax
import jax.numpy as jnp
from jax import lax
from jax.experimental import pallas as pl
from jax.experimental.pallas import tpu as pltpu
from jax.experimental.pallas import tpu_sc as plsc

_NC = 2
_NS = 16
_LANES = 16

_D = 64
_ND = _D // _LANES
_NPHRASE = 3
_L = 20
_K = _NPHRASE * _L

_CB = 8
_SEG = 120
_NSEG = (_CB * _K) // _SEG


def _allsum(v, lane):
    for sh in (1, 2, 4, 8):
        v = v + v.at[lane ^ sh].get(mode="promise_in_bounds")
    return v


def _rsqrt_vec(x):
    bits = lax.bitcast_convert_type(x, jnp.int32)
    y = lax.bitcast_convert_type(jnp.int32(0x5F3759DF) - (bits >> 1), jnp.float32)
    for _ in range(3):
        y = y * (1.5 - 0.5 * x * y * y)
    return y


def _sc_kernel(bpw, idx_hbm, table_hbm, params_hbm, out_hbm,
               idx_v, rows_v, params_v, out_v, sem0, sem1):
    wid = lax.axis_index("s") * _NC + lax.axis_index("c")
    nch = bpw // _CB
    sems = (sem0, sem1)

    pltpu.sync_copy(idx_hbm.at[pl.ds(wid * nch, nch)], idx_v)
    pltpu.sync_copy(params_hbm, params_v)
    tsum = [params_v[0, pl.ds(d * _LANES, _LANES)]
            + params_v[1, pl.ds(d * _LANES, _LANES)]
            + params_v[2, pl.ds(d * _LANES, _LANES)] for d in range(_ND)]
    gam = [params_v[3, pl.ds(d * _LANES, _LANES)] for d in range(_ND)]
    bet = [params_v[4, pl.ds(d * _LANES, _LANES)] for d in range(_ND)]

    def gather_descs(g, buf):
        return [pltpu.make_async_copy(
                    table_hbm.at[idx_v.at[g, s]],
                    rows_v.at[buf, pl.ds(s * _SEG, _SEG)],
                    sems[buf]) for s in range(_NSEG)]

    def issue(g, buf):
        for cp in gather_descs(g, buf):
            cp.start()

    issue(0, 0)

    def outer(g2, carry):
        for par in range(2):
            g = g2 * 2 + par

            @pl.when(g + 1 < nch)
            def _():
                issue(g + 1, 1 - par)

            for cp in gather_descs(g, par):
                cp.wait()

            for b in range(_CB):
                def row_body(j, acc):
                    r = b * _K + j
                    return tuple(acc[d] + rows_v[par, r, pl.ds(d * _LANES, _LANES)]
                                 for d in range(_ND))

                zero = jnp.zeros((_LANES,), jnp.float32)
                acc = lax.fori_loop(0, _K, row_body, (zero,) * _ND, unroll=6)
                y = [acc[d] * (1.0 / _L) + tsum[d] for d in range(_ND)]

                lane = lax.iota(jnp.int32, _LANES)
                s4 = (y[0] + y[1]) + (y[2] + y[3])
                q4 = (y[0] * y[0] + y[1] * y[1]) + (y[2] * y[2] + y[3] * y[3])
                mu = _allsum(s4, lane) * (1.0 / _D)
                var = _allsum(q4, lane) * (1.0 / _D) - mu * mu
                rstd = _rsqrt_vec(var + 1e-5)
                row = g * _CB + b
                for d in range(_ND):
                    gd = rstd * gam[d]
                    out_v[row, pl.ds(d * _LANES, _LANES)] = (
                        (y[d] - mu) * gd + bet[d])
        return carry

    lax.fori_loop(0, nch // 2, outer, 0)
    pltpu.sync_copy(out_v, out_hbm.at[pl.ds(wid * bpw, bpw), :])


def kernel(action_input, arg1_input, arg2_input, emb_table, type_table,
           ln_gamma, ln_beta):
    b = action_input.shape[0]
    nw = _NC * _NS
    bpw = b // nw
    idx = jnp.concatenate([action_input, arg1_input, arg2_input], axis=1)
    idx = idx.astype(jnp.int32).reshape(b // _CB, _NSEG, _SEG)
    params = jnp.concatenate(
        [type_table.astype(jnp.float32),
         ln_gamma.astype(jnp.float32)[None, :],
         ln_beta.astype(jnp.float32)[None, :]], axis=0)

    mesh = plsc.VectorSubcoreMesh(core_axis_name="c", subcore_axis_name="s")
    run = pl.kernel(
        functools.partial(_sc_kernel, bpw),
        mesh=mesh,
        compiler_params=pltpu.CompilerParams(use_tc_tiling_on_sc=False),
        out_type=jax.ShapeDtypeStruct((b, _D), jnp.float32),
        scratch_types=[
            pltpu.VMEM((bpw // _CB, _NSEG, _SEG), jnp.int32),
            pltpu.VMEM((2, _CB * _K, _D), jnp.float32),
            pltpu.VMEM((5, _D), jnp.float32),
            pltpu.VMEM((bpw, _D), jnp.float32),
            pltpu.SemaphoreType.DMA,
            pltpu.SemaphoreType.DMA,
        ],
    )
    return run(idx, emb_table, params)


# speedup vs baseline: 3.7085x; 1.0814x over previous
import functools

import jax
import jax.numpy as jnp
from jax import lax
from jax.experimental import pallas as pl
from jax.experimental.pallas import tpu as pltpu
from jax.experimental.pallas import tpu_sc as plsc

_NC = 2
_NS = 16
_LANES = 16

_D = 64
_ND = _D // _LANES
_NPHRASE = 3
_L = 20
_K = _NPHRASE * _L

_CB = 8
_SEG = 120
_NSEG = (_CB * _K) // _SEG


def _allsum(v, lane):
    for sh in (1, 2, 4, 8):
        v = v + v.at[lane ^ sh].get(mode="promise_in_bounds")
    return v


def _rsqrt_vec(x):
    bits = lax.bitcast_convert_type(x, jnp.int32)
    y = lax.bitcast_convert_type(jnp.int32(0x5F3759DF) - (bits >> 1), jnp.float32)
    for _ in range(3):
        y = y * (1.5 - 0.5 * x * y * y)
    return y


def _sc_kernel(bpw, idx_hbm, table_hbm, params_hbm, out_hbm,
               idx_v, rows_v, params_v, out_v, sem0, sem1):
    wid = lax.axis_index("s") * _NC + lax.axis_index("c")
    nch = bpw // _CB
    sems = (sem0, sem1)

    pltpu.sync_copy(idx_hbm.at[pl.ds(wid * nch, nch)], idx_v)
    pltpu.sync_copy(params_hbm, params_v)
    tsum = [params_v[0, pl.ds(d * _LANES, _LANES)]
            + params_v[1, pl.ds(d * _LANES, _LANES)]
            + params_v[2, pl.ds(d * _LANES, _LANES)] for d in range(_ND)]
    gam = [params_v[3, pl.ds(d * _LANES, _LANES)] for d in range(_ND)]
    bet = [params_v[4, pl.ds(d * _LANES, _LANES)] for d in range(_ND)]

    def gather_descs(g, buf):
        return [pltpu.make_async_copy(
                    table_hbm.at[idx_v.at[g, s]],
                    rows_v.at[buf, pl.ds(s * _SEG, _SEG)],
                    sems[buf]) for s in range(_NSEG)]

    def issue(g, buf):
        for cp in gather_descs(g, buf):
            cp.start()

    issue(0, 0)

    def outer(g2, carry):
        for par in range(2):
            g = g2 * 2 + par

            @pl.when(g + 1 < nch)
            def _():
                issue(g + 1, 1 - par)

            for cp in gather_descs(g, par):
                cp.wait()

            def row_body(j, accs):
                a = list(accs)
                for b in range(_CB):
                    r = b * _K + j
                    for d in range(_ND):
                        a[b * _ND + d] = (a[b * _ND + d]
                                          + rows_v[par, r, pl.ds(d * _LANES, _LANES)])
                return tuple(a)

            zero = jnp.zeros((_LANES,), jnp.float32)
            accs = lax.fori_loop(0, _K, row_body, (zero,) * (_CB * _ND),
                                 unroll=2)

            lane = lax.iota(jnp.int32, _LANES)
            for b in range(_CB):
                acc = accs[b * _ND:(b + 1) * _ND]
                y = [acc[d] * (1.0 / _L) + tsum[d] for d in range(_ND)]

                s4 = (y[0] + y[1]) + (y[2] + y[3])
                q4 = (y[0] * y[0] + y[1] * y[1]) + (y[2] * y[2] + y[3] * y[3])
                mu = _allsum(s4, lane) * (1.0 / _D)
                var = _allsum(q4, lane) * (1.0 / _D) - mu * mu
                rstd = _rsqrt_vec(var + 1e-5)
                row = g * _CB + b
                for d in range(_ND):
                    gd = rstd * gam[d]
                    out_v[row, pl.ds(d * _LANES, _LANES)] = (
                        (y[d] - mu) * gd + bet[d])
        return carry

    lax.fori_loop(0, nch // 2, outer, 0)
    pltpu.sync_copy(out_v, out_hbm.at[pl.ds(wid * bpw, bpw), :])


def kernel(action_input, arg1_input, arg2_input, emb_table, type_table,
           ln_gamma, ln_beta):
    b = action_input.shape[0]
    nw = _NC * _NS
    bpw = b // nw
    idx = jnp.concatenate([action_input, arg1_input, arg2_input], axis=1)
    idx = idx.astype(jnp.int32).reshape(b // _CB, _NSEG, _SEG)
    params = jnp.concatenate(
        [type_table.astype(jnp.float32),
         ln_gamma.astype(jnp.float32)[None, :],
         ln_beta.astype(jnp.float32)[None, :]], axis=0)

    mesh = plsc.VectorSubcoreMesh(core_axis_name="c", subcore_axis_name="s")
    run = pl.kernel(
        functools.partial(_sc_kernel, bpw),
        mesh=mesh,
        compiler_params=pltpu.CompilerParams(use_tc_tiling_on_sc=False),
        out_type=jax.ShapeDtypeStruct((b, _D), jnp.float32),
        scratch_types=[
            pltpu.VMEM((bpw // _CB, _NSEG, _SEG), jnp.int32),
            pltpu.VMEM((2, _CB * _K, _D), jnp.float32),
            pltpu.VMEM((5, _D), jnp.float32),
            pltpu.VMEM((bpw, _D), jnp.float32),
            pltpu.SemaphoreType.DMA,
            pltpu.SemaphoreType.DMA,
        ],
    )
    return run(idx, emb_table, params)
